# baseline (speedup 1.0000x reference)
import jax
import jax.numpy as jnp
from jax.experimental import pallas as pl
from jax.experimental.pallas import tpu as pltpu

SCALE_XY = 10.0
SCALE_WH = 5.0

N, C, A = 64, 81, 8732
CHUNK = 512
SPB = 4


def _one_sample(s, plabel_ref, ploc_ref, gloc_ref, glabel_ref, dboxes_ref,
                dom_ref, out_ref, con_scr):
    iot_c = jax.lax.broadcasted_iota(jnp.int32, (C, 1), 0)

    pos_acc = jnp.zeros((1, CHUNK), jnp.int32)
    sl1_acc = jnp.zeros((1, CHUNK), jnp.float32)
    con_acc = jnp.zeros((1, CHUNK), jnp.float32)
    conm_acc = jnp.zeros((1, CHUNK), jnp.float32)

    for c0 in range(0, A, CHUNK):
        cw = min(CHUNK, A - c0)
        pb = plabel_ref[s, :, c0:c0 + cw]
        gl = glabel_ref[s, :, c0:c0 + cw]

        m = jnp.max(pb, axis=0, keepdims=True)
        e = jnp.exp(pb - m)
        lse = jnp.log(jnp.sum(e, axis=0, keepdims=True))
        picked = jnp.sum(jnp.where(iot_c == gl, pb, 0.0), axis=0,
                         keepdims=True)
        logpt = picked - m - lse
        pt = jnp.exp(logpt)
        omp = 1.0 - pt
        con_ch = -(omp * omp) * logpt
        con_scr[:, c0:c0 + cw] = con_ch

        maskb = gl > 0
        maskf = maskb.astype(jnp.float32)

        plc = ploc_ref[s, :, c0:c0 + cw]
        glc = gloc_ref[s, :, c0:c0 + cw]
        db = dboxes_ref[0, :, c0:c0 + cw]
        gxy = SCALE_XY * (glc[:2] - db[:2]) / db[2:]
        gwh = SCALE_WH * jnp.log(glc[2:] / db[2:])
        d = plc - jnp.concatenate([gxy, gwh], axis=0)
        ad = jnp.abs(d)
        sl1_ch = maskf * jnp.sum(
            jnp.where(ad < 1.0, 0.5 * d * d, ad - 0.5), axis=0,
            keepdims=True)

        pos_ch = maskb.astype(jnp.int32)
        conm_ch = con_ch * maskf
        if cw < CHUNK:
            padc = ((0, 0), (0, CHUNK - cw))
            con_ch = jnp.pad(con_ch, padc)
            conm_ch = jnp.pad(conm_ch, padc)
            pos_ch = jnp.pad(pos_ch, padc)
            sl1_ch = jnp.pad(sl1_ch, padc)
        pos_acc += pos_ch
        con_acc += con_ch
        conm_acc += conm_ch
        sl1_acc += sl1_ch

    pos_i = jnp.sum(pos_acc)
    sl1s = jnp.sum(sl1_acc)
    sum_con = jnp.sum(con_acc)
    sum_con_mask = jnp.sum(conm_acc)

    srcv = (dom_ref[s] == 0).astype(jnp.float32)
    num_mask = (pos_i > 0).astype(jnp.float32)
    posf = jnp.maximum(pos_i.astype(jnp.float32), 1e-06)
    coefv = srcv * (num_mask / posf * (1.0 / N))

    out_ref[...] += (sl1s + sum_con_mask + sum_con) * coefv

    @pl.when(3 * pos_i < A)
    def _():
        con = con_scr[...]
        mask = glabel_ref[s] > 0
        con_neg = jnp.where(mask, 0.0, con) + 0.0
        bits = jax.lax.bitcast_convert_type(con_neg, jnp.int32)
        k = 3 * pos_i

        def vbody(j, t):
            t2 = t | jnp.left_shift(1, 30 - j)
            cnt = jnp.sum((bits >= t2).astype(jnp.int32))
            return jnp.where(cnt >= k, t2, t)

        t = jax.lax.fori_loop(0, 31, vbody, jnp.int32(0))

        cg = jnp.sum((bits > t).astype(jnp.int32))
        tie = bits == t
        mrem = k - cg
        idx = jax.lax.broadcasted_iota(jnp.int32, (1, A), 1)

        def ibody(j, T):
            T2 = T | jnp.left_shift(1, 13 - j)
            cnt = jnp.sum((tie & (idx < T2)).astype(jnp.int32))
            return jnp.where(cnt <= mrem, T2, T)

        T = jax.lax.fori_loop(0, 14, ibody, jnp.int32(0))

        sel = (bits > t) | (tie & (idx < T))
        negsum = jnp.sum(jnp.where(sel, con, 0.0))
        out_ref[...] -= (sum_con - negsum) * coefv


def _body(plabel_ref, ploc_ref, gloc_ref, glabel_ref, dboxes_ref, dom_ref,
          out_ref, con_scr):
    i = pl.program_id(0)

    @pl.when(i == 0)
    def _():
        out_ref[...] = jnp.zeros((1, 128), jnp.float32)

    for s in range(SPB):
        _one_sample(s, plabel_ref, ploc_ref, gloc_ref, glabel_ref,
                    dboxes_ref, dom_ref, out_ref, con_scr)


@jax.jit
def kernel(ploc, plabel, gloc, glabel, domain_label, dboxes):
    glabel3 = glabel.reshape(N, 1, A)
    dom3 = jnp.broadcast_to(domain_label[:, None, None], (N, 1, 128))
    dom3 = dom3.astype(jnp.int32)
    out = pl.pallas_call(
        _body,
        grid=(N // SPB,),
        in_specs=[
            pl.BlockSpec((SPB, C, A), lambda i: (i, 0, 0)),
            pl.BlockSpec((SPB, 4, A), lambda i: (i, 0, 0)),
            pl.BlockSpec((SPB, 4, A), lambda i: (i, 0, 0)),
            pl.BlockSpec((SPB, 1, A), lambda i: (i, 0, 0)),
            pl.BlockSpec((1, 4, A), lambda i: (0, 0, 0)),
            pl.BlockSpec((SPB, 1, 128), lambda i: (i, 0, 0)),
        ],
        out_specs=pl.BlockSpec((1, 128), lambda i: (0, 0)),
        out_shape=jax.ShapeDtypeStruct((1, 128), jnp.float32),
        scratch_shapes=[pltpu.VMEM((1, A), jnp.float32)],
    )(plabel, ploc, gloc, glabel3, dboxes, dom3)
    return out[0, 0]


# speedup vs baseline: 1.0341x; 1.0341x over previous
import jax
import jax.numpy as jnp
from jax.experimental import pallas as pl
from jax.experimental.pallas import tpu as pltpu

SCALE_XY = 10.0
SCALE_WH = 5.0

N, C, A = 64, 81, 8732
CHUNK = 512
SPB = 2


def _one_sample(s, plabel_ref, ploc_ref, gloc_ref, glabel_ref, dboxes_ref,
                dom_ref, out_ref, con_scr):
    iot_c = jax.lax.broadcasted_iota(jnp.int32, (C, 1), 0)

    pos_acc = jnp.zeros((1, CHUNK), jnp.int32)
    sl1_acc = jnp.zeros((1, CHUNK), jnp.float32)
    con_acc = jnp.zeros((1, CHUNK), jnp.float32)
    conm_acc = jnp.zeros((1, CHUNK), jnp.float32)

    for c0 in range(0, A, CHUNK):
        cw = min(CHUNK, A - c0)
        pb = plabel_ref[s, :, c0:c0 + cw]
        gl = glabel_ref[s, :, c0:c0 + cw]

        m = jnp.max(pb, axis=0, keepdims=True)
        e = jnp.exp(pb - m)
        ones_c = jnp.ones((1, C), jnp.float32)
        se = jax.lax.dot_general(ones_c, e, (((1,), (0,)), ((), ())),
                                 preferred_element_type=jnp.float32)
        lse = jnp.log(se)
        picked = jax.lax.dot_general(
            ones_c, jnp.where(iot_c == gl, pb, 0.0),
            (((1,), (0,)), ((), ())), preferred_element_type=jnp.float32)
        logpt = picked - m - lse
        pt = jnp.exp(logpt)
        omp = 1.0 - pt
        con_ch = -(omp * omp) * logpt
        con_scr[:, c0:c0 + cw] = con_ch

        maskb = gl > 0
        maskf = maskb.astype(jnp.float32)

        plc = ploc_ref[s, :, c0:c0 + cw]
        glc = gloc_ref[s, :, c0:c0 + cw]
        db = dboxes_ref[0, :, c0:c0 + cw]
        gxy = SCALE_XY * (glc[:2] - db[:2]) / db[2:]
        gwh = SCALE_WH * jnp.log(glc[2:] / db[2:])
        d = plc - jnp.concatenate([gxy, gwh], axis=0)
        ad = jnp.abs(d)
        sl1_ch = maskf * jnp.sum(
            jnp.where(ad < 1.0, 0.5 * d * d, ad - 0.5), axis=0,
            keepdims=True)

        pos_ch = maskb.astype(jnp.int32)
        conm_ch = con_ch * maskf
        if cw < CHUNK:
            padc = ((0, 0), (0, CHUNK - cw))
            con_ch = jnp.pad(con_ch, padc)
            conm_ch = jnp.pad(conm_ch, padc)
            pos_ch = jnp.pad(pos_ch, padc)
            sl1_ch = jnp.pad(sl1_ch, padc)
        pos_acc += pos_ch
        con_acc += con_ch
        conm_acc += conm_ch
        sl1_acc += sl1_ch

    pos_i = jnp.sum(pos_acc)
    sl1s = jnp.sum(sl1_acc)
    sum_con = jnp.sum(con_acc)
    sum_con_mask = jnp.sum(conm_acc)

    srcv = (dom_ref[s] == 0).astype(jnp.float32)
    num_mask = (pos_i > 0).astype(jnp.float32)
    posf = jnp.maximum(pos_i.astype(jnp.float32), 1e-06)
    coefv = srcv * (num_mask / posf * (1.0 / N))

    out_ref[...] += (sl1s + sum_con_mask + sum_con) * coefv

    @pl.when(3 * pos_i < A)
    def _():
        con = con_scr[...]
        mask = glabel_ref[s] > 0
        con_neg = jnp.where(mask, 0.0, con) + 0.0
        bits = jax.lax.bitcast_convert_type(con_neg, jnp.int32)
        k = 3 * pos_i

        def vbody(j, t):
            t2 = t | jnp.left_shift(1, 30 - j)
            cnt = jnp.sum((bits >= t2).astype(jnp.int32))
            return jnp.where(cnt >= k, t2, t)

        t = jax.lax.fori_loop(0, 31, vbody, jnp.int32(0))

        cg = jnp.sum((bits > t).astype(jnp.int32))
        tie = bits == t
        mrem = k - cg
        idx = jax.lax.broadcasted_iota(jnp.int32, (1, A), 1)

        def ibody(j, T):
            T2 = T | jnp.left_shift(1, 13 - j)
            cnt = jnp.sum((tie & (idx < T2)).astype(jnp.int32))
            return jnp.where(cnt <= mrem, T2, T)

        T = jax.lax.fori_loop(0, 14, ibody, jnp.int32(0))

        sel = (bits > t) | (tie & (idx < T))
        negsum = jnp.sum(jnp.where(sel, con, 0.0))
        out_ref[...] -= (sum_con - negsum) * coefv


def _body(plabel_ref, ploc_ref, gloc_ref, glabel_ref, dboxes_ref, dom_ref,
          out_ref, con_scr):
    i = pl.program_id(0)

    @pl.when(i == 0)
    def _():
        out_ref[...] = jnp.zeros((1, 128), jnp.float32)

    for s in range(SPB):
        _one_sample(s, plabel_ref, ploc_ref, gloc_ref, glabel_ref,
                    dboxes_ref, dom_ref, out_ref, con_scr)


@jax.jit
def kernel(ploc, plabel, gloc, glabel, domain_label, dboxes):
    glabel3 = glabel.reshape(N, 1, A)
    dom3 = jnp.broadcast_to(domain_label[:, None, None], (N, 1, 128))
    dom3 = dom3.astype(jnp.int32)
    out = pl.pallas_call(
        _body,
        grid=(N // SPB,),
        in_specs=[
            pl.BlockSpec((SPB, C, A), lambda i: (i, 0, 0)),
            pl.BlockSpec((SPB, 4, A), lambda i: (i, 0, 0)),
            pl.BlockSpec((SPB, 4, A), lambda i: (i, 0, 0)),
            pl.BlockSpec((SPB, 1, A), lambda i: (i, 0, 0)),
            pl.BlockSpec((1, 4, A), lambda i: (0, 0, 0)),
            pl.BlockSpec((SPB, 1, 128), lambda i: (i, 0, 0)),
        ],
        out_specs=pl.BlockSpec((1, 128), lambda i: (0, 0)),
        out_shape=jax.ShapeDtypeStruct((1, 128), jnp.float32),
        scratch_shapes=[pltpu.VMEM((1, A), jnp.float32)],
    )(plabel, ploc, gloc, glabel3, dboxes, dom3)
    return out[0, 0]
